# Initial kernel scaffold; baseline (speedup 1.0000x reference)
import jax
import jax.numpy as jnp
from jax.experimental import pallas as pl


def kernel(token_embedding, pos_table):
    raise NotImplementedError("write your pallas kernel here")


# speedup vs baseline: 1.6943x; 1.6943x over previous
import jax
import jax.numpy as jnp
from jax.experimental import pallas as pl


def _add_kernel(te_ref, pos_ref, out_ref):
    out_ref[...] = te_ref[...] + pos_ref[...]


def kernel(token_embedding, pos_table):
    B, S, E = token_embedding.shape
    BS = 512
    grid = (S // BS, B)
    return pl.pallas_call(
        _add_kernel,
        grid=grid,
        in_specs=[
            pl.BlockSpec((1, BS, E), lambda i, j: (j, i, 0)),
            pl.BlockSpec((BS, E), lambda i, j: (i, 0)),
        ],
        out_specs=pl.BlockSpec((1, BS, E), lambda i, j: (j, i, 0)),
        out_shape=jax.ShapeDtypeStruct((B, S, E), token_embedding.dtype),
    )(token_embedding, pos_table)


# speedup vs baseline: 1.8719x; 1.1048x over previous
import jax
import jax.numpy as jnp
from jax.experimental import pallas as pl


def _add_kernel(te_ref, pos_ref, out_ref):
    out_ref[...] = te_ref[...] + pos_ref[...]


def kernel(token_embedding, pos_table):
    B, S, E = token_embedding.shape
    BS = 1024
    grid = (S // BS, B)
    return pl.pallas_call(
        _add_kernel,
        grid=grid,
        in_specs=[
            pl.BlockSpec((1, BS, E), lambda i, j: (j, i, 0)),
            pl.BlockSpec((BS, E), lambda i, j: (i, 0)),
        ],
        out_specs=pl.BlockSpec((1, BS, E), lambda i, j: (j, i, 0)),
        out_shape=jax.ShapeDtypeStruct((B, S, E), token_embedding.dtype),
    )(token_embedding, pos_table)


# speedup vs baseline: 1.9960x; 1.0663x over previous
import jax
import jax.numpy as jnp
from jax.experimental import pallas as pl


def _add_kernel(te_ref, pos_ref, out_ref):
    out_ref[...] = te_ref[...] + pos_ref[...]


def kernel(token_embedding, pos_table):
    B, S, E = token_embedding.shape
    BS = 2048
    grid = (S // BS, B)
    return pl.pallas_call(
        _add_kernel,
        grid=grid,
        in_specs=[
            pl.BlockSpec((1, BS, E), lambda i, j: (j, i, 0)),
            pl.BlockSpec((BS, E), lambda i, j: (i, 0)),
        ],
        out_specs=pl.BlockSpec((1, BS, E), lambda i, j: (j, i, 0)),
        out_shape=jax.ShapeDtypeStruct((B, S, E), token_embedding.dtype),
    )(token_embedding, pos_table)
